# baseline (speedup 1.0000x reference)
import jax
import jax.numpy as jnp
from jax import lax
from jax.experimental import pallas as pl
from jax.experimental.pallas import tpu as pltpu
from jax.experimental.pallas import tpu_sc as plsc

N_ENT = 1000000
N_REL = 1000
RANK = 32
B = 16384

NC = 2
NS = 16
NW = NC * NS
BPW = B // NW
CH = 128
NCH = BPW // CH
L = 16


def _sc_body(h_hbm, r_hbm, t_hbm, entity_hbm, rel_hbm, bh_hbm, bt_hbm,
             pred_out, head_out, rele_out, tail_out,
             idxh_v, idxr_v, idxt_v, hrows, rrows, trows, bhv, btv, pred_v,
             gsems, osem):
    wid = lax.axis_index("s") * NC + lax.axis_index("c")
    base = wid * BPW
    rbase = wid * NCH

    pltpu.sync_copy(h_hbm.at[pl.ds(rbase, NCH)], idxh_v)
    pltpu.sync_copy(r_hbm.at[pl.ds(rbase, NCH)], idxr_v)
    pltpu.sync_copy(t_hbm.at[pl.ds(rbase, NCH)], idxt_v)

    copies = []
    for j in range(NCH):
        dst = pl.ds(j * CH, CH)
        sem = gsems.at[j]
        cj = [
            pltpu.async_copy(entity_hbm.at[idxh_v.at[j]], hrows.at[dst], sem),
            pltpu.async_copy(entity_hbm.at[idxt_v.at[j]], trows.at[dst], sem),
            pltpu.async_copy(rel_hbm.at[idxr_v.at[j]], rrows.at[dst], sem),
            pltpu.async_copy(bh_hbm.at[idxh_v.at[j]], bhv.at[dst], sem),
            pltpu.async_copy(bt_hbm.at[idxt_v.at[j]], btv.at[dst], sem),
        ]
        copies.append(cj)

    lane = lax.iota(jnp.int32, L)
    masks = [(lane >> k) % 2 == 0 for k in range(4)]
    perms = [lane ^ (1 << k) for k in range(4)]
    gdn = lax.GatherDimensionNumbers(
        offset_dims=(), collapsed_slice_dims=(0,), start_index_map=(0,))

    def shuf(v, perm):
        return lax.gather(v, perm[:, None], gdn, slice_sizes=(1,),
                          mode=lax.GatherScatterMode.PROMISE_IN_BOUNDS)

    def group(g, carry):
        vs = []
        for j in range(L):
            q = g * L + j
            h0 = hrows[q, pl.ds(0, L)]
            h1 = hrows[q, pl.ds(L, L)]
            r0 = rrows[q, pl.ds(0, L)]
            r1 = rrows[q, pl.ds(L, L)]
            t0 = trows[q, pl.ds(0, L)]
            t1 = trows[q, pl.ds(L, L)]
            d0 = h0 + r0 - t0
            d1 = h1 + r1 - t1
            vs.append(d0 * d0 + d1 * d1)
        for k in range(4):
            m, p = masks[k], perms[k]
            vs = [jnp.where(m, a, b) + shuf(jnp.where(m, b, a), p)
                  for a, b in zip(vs[0::2], vs[1::2])]
        gb = pl.ds(g * L, L)
        pred_v[gb] = bhv[gb] + btv[gb] - vs[0]
        return carry

    out_copies = []
    gpc = CH // L
    for j in range(NCH):
        for c in copies[j]:
            c.wait()
        lax.fori_loop(j * gpc, (j + 1) * gpc, group, 0)
        cb = pl.ds(j * CH, CH)
        hb = pl.ds(base + j * CH, CH)
        out_copies += [
            pltpu.async_copy(hrows.at[cb], head_out.at[hb], osem),
            pltpu.async_copy(rrows.at[cb], rele_out.at[hb], osem),
            pltpu.async_copy(trows.at[cb], tail_out.at[hb], osem),
        ]
    out_copies.append(
        pltpu.async_copy(pred_v, pred_out.at[pl.ds(base, BPW)], osem))
    for c in out_copies:
        c.wait()


@jax.jit
def _run(h2, r2, t2, entity, rel, bh1, bt1):
    mesh = plsc.VectorSubcoreMesh(core_axis_name="c", subcore_axis_name="s",
                                  num_cores=NC, num_subcores=NS)
    k = pl.kernel(
        _sc_body,
        out_type=(
            jax.ShapeDtypeStruct((B,), jnp.float32),
            jax.ShapeDtypeStruct((B, RANK), jnp.float32),
            jax.ShapeDtypeStruct((B, RANK), jnp.float32),
            jax.ShapeDtypeStruct((B, RANK), jnp.float32),
        ),
        mesh=mesh,
        scratch_types=[
            pltpu.VMEM((NCH, CH), jnp.int32),
            pltpu.VMEM((NCH, CH), jnp.int32),
            pltpu.VMEM((NCH, CH), jnp.int32),
            pltpu.VMEM((BPW, RANK), jnp.float32),
            pltpu.VMEM((BPW, RANK), jnp.float32),
            pltpu.VMEM((BPW, RANK), jnp.float32),
            pltpu.VMEM((BPW,), jnp.float32),
            pltpu.VMEM((BPW,), jnp.float32),
            pltpu.VMEM((BPW,), jnp.float32),
            pltpu.SemaphoreType.DMA((NCH,)),
            pltpu.SemaphoreType.DMA,
        ],
        compiler_params=pltpu.CompilerParams(use_tc_tiling_on_sc=False),
    )
    return k(h2, r2, t2, entity, rel, bh1, bt1)


def kernel(queries, entity, rel, bh, bt):
    h2 = queries[:, 0].reshape(B // CH, CH)
    r2 = queries[:, 1].reshape(B // CH, CH)
    t2 = queries[:, 2].reshape(B // CH, CH)
    bh1 = bh.reshape(-1)
    bt1 = bt.reshape(-1)
    pred, head_e, rel_e, rhs_e = _run(h2, r2, t2, entity, rel, bh1, bt1)
    return pred.reshape(B, 1), head_e, rel_e, rhs_e
